# Initial kernel scaffold; baseline (speedup 1.0000x reference)
import jax
import jax.numpy as jnp
from jax.experimental import pallas as pl


def kernel(x, edge_index, Wl1, Wr1, b1, Wl2, Wr2, b2):
    raise NotImplementedError("write your pallas kernel here")


# speedup vs baseline: 7.4119x; 7.4119x over previous
import functools

import jax
import jax.numpy as jnp
from jax import lax
from jax.experimental import pallas as pl
from jax.experimental.pallas import tpu as pltpu
from jax.experimental.pallas import tpu_sc as plsc

N_NODES = 10000
N_EDGES = 320000
IN_DIM = 128
HID_DIM = 64
OUT_DIM = 64

NC = 2
NS = 16
NW = NC * NS
EB = 128
N_ROWS = N_EDGES // EB
N_PAD = 10240
ROWS_PER_TILE = N_PAD // NS
CNT_W = 16


def _sc_mesh():
    return plsc.VectorSubcoreMesh(
        core_axis_name="c", subcore_axis_name="s",
        num_cores=NC, num_subcores=NS)


def _sc_agg_body(want_counts, *refs):
    if want_counts:
        (table, srcr, dstr, z64, z16, ones_hbm,
         agg_out, cnt_out,
         acc_sh, cnt_sh, src_v, dst_v, rows_v, ones_v, sem) = refs
    else:
        (table, srcr, dstr, z64,
         agg_out,
         acc_sh, src_v, dst_v, rows_v, sem) = refs

    c = lax.axis_index("c")
    s = lax.axis_index("s")
    wid = c * NS + s

    sl = pl.ds(s * ROWS_PER_TILE, ROWS_PER_TILE)
    pltpu.sync_copy(z64.at[sl], acc_sh.at[sl])
    if want_counts:
        pltpu.sync_copy(z16.at[sl], cnt_sh.at[sl])
        pltpu.sync_copy(ones_hbm, ones_v)
    plsc.subcore_barrier()

    n_t = (N_ROWS + NW - 1) // NW

    def step(t, carry):
        r = wid + NW * t

        @pl.when(r < N_ROWS)
        def _():
            pltpu.sync_copy(srcr.at[r], src_v)
            pltpu.sync_copy(dstr.at[r], dst_v)
            pltpu.async_copy(table.at[src_v], rows_v, sem).wait()
            pltpu.sync_copy(rows_v, acc_sh.at[dst_v], add=True)
            if want_counts:
                pltpu.sync_copy(ones_v, cnt_sh.at[dst_v], add=True)

        return carry

    lax.fori_loop(0, n_t, step, 0)
    plsc.subcore_barrier()

    pltpu.sync_copy(acc_sh.at[sl], agg_out.at[c, sl])
    if want_counts:
        pltpu.sync_copy(cnt_sh.at[sl], cnt_out.at[c, sl])


def _make_sc_agg(want_counts):
    out_type = [jax.ShapeDtypeStruct((NC, N_PAD, HID_DIM), jnp.float32)]
    scratch = [
        pltpu.VMEM_SHARED((N_PAD, HID_DIM), jnp.float32),
    ]
    if want_counts:
        out_type.append(jax.ShapeDtypeStruct((NC, N_PAD, CNT_W), jnp.float32))
        scratch.append(pltpu.VMEM_SHARED((N_PAD, CNT_W), jnp.float32))
    scratch += [
        pltpu.VMEM((EB,), jnp.int32),
        pltpu.VMEM((EB,), jnp.int32),
        pltpu.VMEM((EB, HID_DIM), jnp.float32),
    ]
    if want_counts:
        scratch.append(pltpu.VMEM((EB, CNT_W), jnp.float32))
    scratch.append(pltpu.SemaphoreType.DMA)

    return pl.kernel(
        functools.partial(_sc_agg_body, want_counts),
        out_type=out_type,
        mesh=_sc_mesh(),
        scratch_types=scratch,
        compiler_params=pltpu.CompilerParams(use_tc_tiling_on_sc=False),
    )


def _tc_pre(x, Wl1, Wr1):
    def body(x_ref, wl_ref, wr_ref, xl_ref, xr_ref):
        xb = x_ref[...]
        xl_ref[...] = jnp.dot(xb, wl_ref[...], preferred_element_type=jnp.float32)
        xr_ref[...] = jnp.dot(xb, wr_ref[...], preferred_element_type=jnp.float32)

    blk = 1000
    return pl.pallas_call(
        body,
        grid=(N_NODES // blk,),
        in_specs=[
            pl.BlockSpec((blk, IN_DIM), lambda i: (i, 0)),
            pl.BlockSpec((IN_DIM, HID_DIM), lambda i: (0, 0)),
            pl.BlockSpec((IN_DIM, HID_DIM), lambda i: (0, 0)),
        ],
        out_specs=[pl.BlockSpec((blk, HID_DIM), lambda i: (i, 0))] * 2,
        out_shape=[jax.ShapeDtypeStruct((N_NODES, HID_DIM), jnp.float32)] * 2,
    )(x, Wl1, Wr1)


def _tc_mid(aggp, cntp, xr, b1, Wl2, Wr2, b2):
    def body(aggp_ref, cntp_ref, xr_ref, b1_ref, wl2_ref, wr2_ref, b2_ref,
             hl_ref, hb_ref):
        cnt = cntp_ref[0][:, 0:1] + cntp_ref[1][:, 0:1]
        inv = 1.0 / jnp.maximum(cnt, 1.0)
        mean = (aggp_ref[0] + aggp_ref[1]) * inv
        h = jnp.maximum(mean + b1_ref[...] + xr_ref[...], 0.0)
        hl_ref[...] = jnp.dot(h, wl2_ref[...], preferred_element_type=jnp.float32)
        hb_ref[...] = (jnp.dot(h, wr2_ref[...], preferred_element_type=jnp.float32)
                       + b2_ref[...])

    blk = 1000
    return pl.pallas_call(
        body,
        grid=(N_NODES // blk,),
        in_specs=[
            pl.BlockSpec((NC, blk, HID_DIM), lambda i: (0, i, 0)),
            pl.BlockSpec((NC, blk, CNT_W), lambda i: (0, i, 0)),
            pl.BlockSpec((blk, HID_DIM), lambda i: (i, 0)),
            pl.BlockSpec((1, HID_DIM), lambda i: (0, 0)),
            pl.BlockSpec((HID_DIM, OUT_DIM), lambda i: (0, 0)),
            pl.BlockSpec((HID_DIM, OUT_DIM), lambda i: (0, 0)),
            pl.BlockSpec((1, OUT_DIM), lambda i: (0, 0)),
        ],
        out_specs=[pl.BlockSpec((blk, OUT_DIM), lambda i: (i, 0))] * 2,
        out_shape=[jax.ShapeDtypeStruct((N_NODES, OUT_DIM), jnp.float32)] * 2,
    )(aggp, cntp, xr, b1, Wl2, Wr2, b2)


def _tc_fin(aggp2, cntp, hb):
    def body(aggp_ref, cntp_ref, hb_ref, out_ref):
        cnt = cntp_ref[0][:, 0:1] + cntp_ref[1][:, 0:1]
        inv = 1.0 / jnp.maximum(cnt, 1.0)
        out_ref[...] = (aggp_ref[0] + aggp_ref[1]) * inv + hb_ref[...]

    blk = 1000
    return pl.pallas_call(
        body,
        grid=(N_NODES // blk,),
        in_specs=[
            pl.BlockSpec((NC, blk, OUT_DIM), lambda i: (0, i, 0)),
            pl.BlockSpec((NC, blk, CNT_W), lambda i: (0, i, 0)),
            pl.BlockSpec((blk, OUT_DIM), lambda i: (i, 0)),
        ],
        out_specs=pl.BlockSpec((blk, OUT_DIM), lambda i: (i, 0)),
        out_shape=jax.ShapeDtypeStruct((N_NODES, OUT_DIM), jnp.float32),
    )(aggp2, cntp, hb)


def kernel(x, edge_index, Wl1, Wr1, b1, Wl2, Wr2, b2):
    ei = edge_index.astype(jnp.int32)
    srcr = ei[0].reshape(N_ROWS, EB)
    dstr = ei[1].reshape(N_ROWS, EB)
    z64 = jnp.zeros((N_PAD, HID_DIM), jnp.float32)
    z16 = jnp.zeros((N_PAD, CNT_W), jnp.float32)
    ones = jnp.ones((EB, CNT_W), jnp.float32)

    xl, xr = _tc_pre(x, Wl1, Wr1)
    aggp, cntp = _make_sc_agg(True)(xl, srcr, dstr, z64, z16, ones)
    hl, hb = _tc_mid(aggp, cntp, xr, b1.reshape(1, -1), Wl2, Wr2,
                     b2.reshape(1, -1))
    (aggp2,) = _make_sc_agg(False)(hl, srcr, dstr, z64)
    return _tc_fin(aggp2, cntp, hb)


# speedup vs baseline: 14.3160x; 1.9315x over previous
import functools

import jax
import jax.numpy as jnp
from jax import lax
from jax.experimental import pallas as pl
from jax.experimental.pallas import tpu as pltpu
from jax.experimental.pallas import tpu_sc as plsc

N_NODES = 10000
N_EDGES = 320000
IN_DIM = 128
HID_DIM = 64
OUT_DIM = 64

NC = 2
NS = 16
NW = NC * NS
EB = 128
N_ROWS = 2560
RPW = N_ROWS // NW
N_PAD = 10240
ROWS_PER_TILE = N_PAD // NS
CNT_W = 16
E_PAD = N_ROWS * EB - N_EDGES


def _sc_mesh():
    return plsc.VectorSubcoreMesh(
        core_axis_name="c", subcore_axis_name="s",
        num_cores=NC, num_subcores=NS)


def _sc_agg_body(want_counts, *refs):
    if want_counts:
        (table, srcr, dstr, z64, z16, ones_hbm,
         agg_out, cnt_out,
         acc_sh, cnt_sh, srcb, dstb, rows0, rows1, ones_v,
         sem0, sem1) = refs
    else:
        (table, srcr, dstr, z64,
         agg_out,
         acc_sh, srcb, dstb, rows0, rows1, sem0, sem1) = refs

    c = lax.axis_index("c")
    s = lax.axis_index("s")
    wid = c * NS + s
    rbase = wid * RPW

    pltpu.sync_copy(srcr.at[pl.ds(rbase, RPW)], srcb)
    pltpu.sync_copy(dstr.at[pl.ds(rbase, RPW)], dstb)

    sl = pl.ds(s * ROWS_PER_TILE, ROWS_PER_TILE)
    pltpu.sync_copy(z64.at[sl], acc_sh.at[sl])
    if want_counts:
        pltpu.sync_copy(z16.at[sl], cnt_sh.at[sl])
        pltpu.sync_copy(ones_hbm, ones_v)
    plsc.subcore_barrier()

    pltpu.async_copy(table.at[srcb.at[0]], rows0, sem0)
    pltpu.async_copy(table.at[srcb.at[1]], rows1, sem1)

    def half(i, t, rows_v, sem):
        pltpu.make_async_copy(table.at[srcb.at[t]], rows_v, sem).wait()
        pltpu.sync_copy(rows_v, acc_sh.at[dstb.at[t]], add=True)
        if want_counts:
            pltpu.sync_copy(ones_v, cnt_sh.at[dstb.at[t]], add=True)

        @pl.when(i < RPW // 2 - 1)
        def _():
            pltpu.async_copy(table.at[srcb.at[t + 2]], rows_v, sem)

    def pair(i, carry):
        half(i, 2 * i, rows0, sem0)
        half(i, 2 * i + 1, rows1, sem1)
        return carry

    lax.fori_loop(0, RPW // 2, pair, 0)
    plsc.subcore_barrier()

    pltpu.sync_copy(acc_sh.at[sl], agg_out.at[c, sl])
    if want_counts:
        pltpu.sync_copy(cnt_sh.at[sl], cnt_out.at[c, sl])


def _make_sc_agg(want_counts):
    out_type = [jax.ShapeDtypeStruct((NC, N_PAD, HID_DIM), jnp.float32)]
    scratch = [
        pltpu.VMEM_SHARED((N_PAD, HID_DIM), jnp.float32),
    ]
    if want_counts:
        out_type.append(jax.ShapeDtypeStruct((NC, N_PAD, CNT_W), jnp.float32))
        scratch.append(pltpu.VMEM_SHARED((N_PAD, CNT_W), jnp.float32))
    scratch += [
        pltpu.VMEM((RPW, EB), jnp.int32),
        pltpu.VMEM((RPW, EB), jnp.int32),
        pltpu.VMEM((EB, HID_DIM), jnp.float32),
        pltpu.VMEM((EB, HID_DIM), jnp.float32),
    ]
    if want_counts:
        scratch.append(pltpu.VMEM((EB, CNT_W), jnp.float32))
    scratch += [pltpu.SemaphoreType.DMA, pltpu.SemaphoreType.DMA]

    return pl.kernel(
        functools.partial(_sc_agg_body, want_counts),
        out_type=out_type,
        mesh=_sc_mesh(),
        scratch_types=scratch,
        compiler_params=pltpu.CompilerParams(use_tc_tiling_on_sc=False),
    )


def _tc_pre(x, Wl1, Wr1):
    def body(x_ref, wl_ref, wr_ref, xl_ref, xr_ref):
        xb = x_ref[...]
        xl_ref[...] = jnp.dot(xb, wl_ref[...], preferred_element_type=jnp.float32)
        xr_ref[...] = jnp.dot(xb, wr_ref[...], preferred_element_type=jnp.float32)

    blk = 1000
    return pl.pallas_call(
        body,
        grid=(N_NODES // blk,),
        in_specs=[
            pl.BlockSpec((blk, IN_DIM), lambda i: (i, 0)),
            pl.BlockSpec((IN_DIM, HID_DIM), lambda i: (0, 0)),
            pl.BlockSpec((IN_DIM, HID_DIM), lambda i: (0, 0)),
        ],
        out_specs=[pl.BlockSpec((blk, HID_DIM), lambda i: (i, 0))] * 2,
        out_shape=[jax.ShapeDtypeStruct((N_NODES, HID_DIM), jnp.float32)] * 2,
    )(x, Wl1, Wr1)


def _tc_mid(aggp, cntp, xr, b1, Wl2, Wr2, b2):
    def body(aggp_ref, cntp_ref, xr_ref, b1_ref, wl2_ref, wr2_ref, b2_ref,
             hl_ref, hb_ref):
        cnt = cntp_ref[0][:, 0:1] + cntp_ref[1][:, 0:1]
        inv = 1.0 / jnp.maximum(cnt, 1.0)
        mean = (aggp_ref[0] + aggp_ref[1]) * inv
        h = jnp.maximum(mean + b1_ref[...] + xr_ref[...], 0.0)
        hl_ref[...] = jnp.dot(h, wl2_ref[...], preferred_element_type=jnp.float32)
        hb_ref[...] = (jnp.dot(h, wr2_ref[...], preferred_element_type=jnp.float32)
                       + b2_ref[...])

    blk = 1000
    return pl.pallas_call(
        body,
        grid=(N_NODES // blk,),
        in_specs=[
            pl.BlockSpec((NC, blk, HID_DIM), lambda i: (0, i, 0)),
            pl.BlockSpec((NC, blk, CNT_W), lambda i: (0, i, 0)),
            pl.BlockSpec((blk, HID_DIM), lambda i: (i, 0)),
            pl.BlockSpec((1, HID_DIM), lambda i: (0, 0)),
            pl.BlockSpec((HID_DIM, OUT_DIM), lambda i: (0, 0)),
            pl.BlockSpec((HID_DIM, OUT_DIM), lambda i: (0, 0)),
            pl.BlockSpec((1, OUT_DIM), lambda i: (0, 0)),
        ],
        out_specs=[pl.BlockSpec((blk, OUT_DIM), lambda i: (i, 0))] * 2,
        out_shape=[jax.ShapeDtypeStruct((N_NODES, OUT_DIM), jnp.float32)] * 2,
    )(aggp, cntp, xr, b1, Wl2, Wr2, b2)


def _tc_fin(aggp2, cntp, hb):
    def body(aggp_ref, cntp_ref, hb_ref, out_ref):
        cnt = cntp_ref[0][:, 0:1] + cntp_ref[1][:, 0:1]
        inv = 1.0 / jnp.maximum(cnt, 1.0)
        out_ref[...] = (aggp_ref[0] + aggp_ref[1]) * inv + hb_ref[...]

    blk = 1000
    return pl.pallas_call(
        body,
        grid=(N_NODES // blk,),
        in_specs=[
            pl.BlockSpec((NC, blk, OUT_DIM), lambda i: (0, i, 0)),
            pl.BlockSpec((NC, blk, CNT_W), lambda i: (0, i, 0)),
            pl.BlockSpec((blk, OUT_DIM), lambda i: (i, 0)),
        ],
        out_specs=pl.BlockSpec((blk, OUT_DIM), lambda i: (i, 0)),
        out_shape=jax.ShapeDtypeStruct((N_NODES, OUT_DIM), jnp.float32),
    )(aggp2, cntp, hb)


def kernel(x, edge_index, Wl1, Wr1, b1, Wl2, Wr2, b2):
    ei = edge_index.astype(jnp.int32)
    pad_src = jnp.arange(E_PAD, dtype=jnp.int32) % N_NODES
    pad_dst = N_NODES + (jnp.arange(E_PAD, dtype=jnp.int32) % (N_PAD - N_NODES))
    srcr = jnp.concatenate([ei[0], pad_src]).reshape(N_ROWS, EB)
    dstr = jnp.concatenate([ei[1], pad_dst]).reshape(N_ROWS, EB)
    z64 = jnp.zeros((N_PAD, HID_DIM), jnp.float32)
    z16 = jnp.zeros((N_PAD, CNT_W), jnp.float32)
    ones = jnp.ones((EB, CNT_W), jnp.float32)

    xl, xr = _tc_pre(x, Wl1, Wr1)
    aggp, cntp = _make_sc_agg(True)(xl, srcr, dstr, z64, z16, ones)
    hl, hb = _tc_mid(aggp, cntp, xr, b1.reshape(1, -1), Wl2, Wr2,
                     b2.reshape(1, -1))
    (aggp2,) = _make_sc_agg(False)(hl, srcr, dstr, z64)
    return _tc_fin(aggp2, cntp, hb)
